# Initial kernel scaffold; baseline (speedup 1.0000x reference)
import jax
import jax.numpy as jnp
from jax.experimental import pallas as pl


def kernel(x, edge_index, edge_weight, W1, b1, W_mu, b_mu, W_ls, b_ls):
    raise NotImplementedError("write your pallas kernel here")


# speedup vs baseline: 33.9879x; 33.9879x over previous
import functools

import jax
import jax.numpy as jnp
from jax import lax
from jax.experimental import pallas as pl
from jax.experimental.pallas import tpu as pltpu
from jax.experimental.pallas import tpu_sc as plsc

N = 10000
E = 320000
C = 128
NC, NS = 2, 16
NW = NC * NS
EPW = 10240
EPAD = NW * EPW
K = 64
NCH = EPW // K
NB = 5
NP = 10240
DK = 128
DNCH = EPW // DK
NPR = 10240
RPT = NPR // NS

@functools.lru_cache(maxsize=None)
def _get_mesh():
    return plsc.VectorSubcoreMesh(
        core_axis_name="c", subcore_axis_name="s", num_cores=NC, num_subcores=NS
    )


@functools.lru_cache(maxsize=None)
def _get_sc_degree():
    return functools.partial(
        pl.kernel,
        out_type=jax.ShapeDtypeStruct((NC, NP), jnp.float32),
        mesh=_get_mesh(),
        scratch_types=[
            pltpu.VMEM((DNCH, DK), jnp.int32),
            pltpu.VMEM((DNCH, DK), jnp.float32),
            pltpu.VMEM((NP // NS,), jnp.float32),
            pltpu.MemorySpace.VMEM_SHARED((NP,), jnp.float32),
        ],
    )(_sc_degree_body)


def _sc_degree_body(dst_hbm, ew_hbm, out_hbm, dst_v, ew_v, zero_v, acc_sh):
    c = lax.axis_index("c")
    s = lax.axis_index("s")
    wid = c * NS + s
    zvec = jnp.zeros((16,), jnp.float32)

    def _zero(i, carry):
        zero_v[pl.ds(i * 16, 16)] = zvec
        return carry

    lax.fori_loop(0, (NP // NS) // 16, _zero, 0)
    pltpu.sync_copy(zero_v, acc_sh.at[pl.ds(s * (NP // NS), NP // NS)])
    pltpu.sync_copy(dst_hbm.at[wid], dst_v)
    pltpu.sync_copy(ew_hbm.at[wid], ew_v)
    plsc.subcore_barrier()

    def _chunk(ci, carry):
        pltpu.sync_copy(ew_v.at[ci], acc_sh.at[dst_v.at[ci]], add=True)
        return carry

    lax.fori_loop(0, DNCH, _chunk, 0)
    plsc.subcore_barrier()
    pltpu.sync_copy(
        acc_sh.at[pl.ds(s * (NP // NS), NP // NS)],
        out_hbm.at[c, pl.ds(s * (NP // NS), NP // NS)],
    )


@functools.lru_cache(maxsize=None)
def _get_sc_aggregate():
    return functools.partial(
        pl.kernel,
        out_type=jax.ShapeDtypeStruct((NC, NPR, C), jnp.float32),
        mesh=_get_mesh(),
        scratch_types=(
            [
                pltpu.VMEM((NB, 2, K), jnp.int32),
                pltpu.VMEM((NB, K), jnp.float32),
            ]
            + [pltpu.VMEM((K, C), jnp.float32) for _ in range(NB)]
            + [pltpu.MemorySpace.VMEM_SHARED((NPR, C), jnp.float32)]
            + [pltpu.SemaphoreType.DMA for _ in range(3 * NB)]
        ),
    )(_sc_aggregate_body)


_DNUMS = lax.GatherDimensionNumbers(
    offset_dims=(), collapsed_slice_dims=(0,), start_index_map=(0,)
)


def _sc_aggregate_body(g_hbm, pk_hbm, ew_hbm, out_hbm, pk_v, ew_r, *rest):
    rvs = list(rest[:NB])
    acc_sh = rest[NB]
    spk = list(rest[NB + 1 : NB + 1 + NB])
    sg = list(rest[NB + 1 + NB : NB + 1 + 2 * NB])
    ss = list(rest[NB + 1 + 2 * NB : NB + 1 + 3 * NB])
    rv0 = rvs[0]
    c = lax.axis_index("c")
    s = lax.axis_index("s")
    wid = c * NS + s
    zvec = jnp.zeros((16,), jnp.float32)

    def _zero(i, carry):
        for k in range(C // 16):
            rv0[i, pl.ds(k * 16, 16)] = zvec
        return carry

    lax.fori_loop(0, K, _zero, 0)
    for t in range(RPT // K):
        pltpu.sync_copy(rv0, acc_sh.at[pl.ds(s * RPT + t * K, K)])
    plsc.subcore_barrier()

    def _issue_idx(ci, slot):
        pltpu.async_copy(pk_hbm.at[wid, ci], pk_v.at[slot], spk[slot])
        pltpu.async_copy(ew_hbm.at[wid, ci], ew_r.at[slot], spk[slot])

    def _wait_idx(ci, slot):
        pltpu.make_async_copy(
            pk_hbm.at[wid, ci], pk_v.at[slot], spk[slot]
        ).wait()
        pltpu.make_async_copy(
            ew_hbm.at[wid, ci], ew_r.at[slot], spk[slot]
        ).wait()

    _issue_idx(0, 0)
    _wait_idx(0, 0)
    pltpu.async_copy(g_hbm.at[pk_v.at[0, 0]], rv0, sg[0])
    _issue_idx(1, 1)

    def _outer(oi, carry):
        for sl in range(NB):
            ci = oi * NB + sl
            sn1 = (sl + 1) % NB
            sn2 = (sl + 2) % NB
            rv = rvs[sl]

            @pl.when(ci >= 3)
            def _drain_scatter():
                pltpu.make_async_copy(
                    rvs[sn2], acc_sh.at[pk_v.at[sn2, 1]], ss[sn2]
                ).wait()

            @pl.when(ci + 1 < NCH)
            def _issue_next_gather():
                _wait_idx(ci + 1, sn1)
                pltpu.async_copy(g_hbm.at[pk_v.at[sn1, 0]], rvs[sn1], sg[sn1])

            @pl.when(ci + 2 < NCH)
            def _prefetch_idx():
                _issue_idx(ci + 2, sn2)

            pltpu.make_async_copy(g_hbm.at[pk_v.at[sl, 0]], rv, sg[sl]).wait()

            def _group(gi, carry2):
                ew16 = ew_r[sl, pl.ds(gi * 16, 16)]
                for j in range(16):
                    ewj = lax.gather(
                        ew16, jnp.full((16, 1), j, jnp.int32), _DNUMS, (1,),
                        mode=lax.GatherScatterMode.PROMISE_IN_BOUNDS,
                    )
                    row = gi * 16 + j
                    for k in range(C // 16):
                        rv[row, pl.ds(k * 16, 16)] = (
                            rv[row, pl.ds(k * 16, 16)] * ewj
                        )
                return carry2

            lax.fori_loop(0, K // 16, _group, 0)
            pltpu.make_async_copy(
                rv, acc_sh.at[pk_v.at[sl, 1]], ss[sl]
            ).start(add=True)
        return carry

    lax.fori_loop(0, NCH // NB, _outer, 0)
    for last in (NCH - 3, NCH - 2, NCH - 1):
        sl = last % NB
        pltpu.make_async_copy(rvs[sl], acc_sh.at[pk_v.at[sl, 1]], ss[sl]).wait()
    plsc.subcore_barrier()
    for t in range(RPT // 128):
        pltpu.sync_copy(
            acc_sh.at[pl.ds(s * RPT + t * 128, 128)],
            out_hbm.at[c, pl.ds(s * RPT + t * 128, 128)],
        )


_R = 2000


def _tc1_body(degp_ref, x_ref, w_ref, g0_ref, dis_ref):
    deg = 1.0 + degp_ref[0] + degp_ref[1]
    dis = lax.rsqrt(deg)
    z = jnp.dot(x_ref[...], w_ref[...], preferred_element_type=jnp.float32)
    g0_ref[...] = z * dis
    dis_ref[...] = dis


def _tc2_body(p_ref, g0_ref, dis_ref, b_ref, w_ref, g1_ref):
    dis = dis_ref[...]
    h1 = jnp.maximum(dis * (p_ref[0] + p_ref[1] + g0_ref[...]) + b_ref[...], 0.0)
    m = jnp.dot(h1, w_ref[...], preferred_element_type=jnp.float32)
    g1_ref[...] = m * dis


def _tc3_body(q_ref, g1_ref, dis_ref, b_ref, mu_ref, ls_ref):
    out = dis_ref[...] * (q_ref[0] + q_ref[1] + g1_ref[...]) + b_ref[...]
    mu_ref[...] = out[:, : C // 2]
    ls_ref[...] = out[:, C // 2 :]


def kernel(x, edge_index, edge_weight, W1, b1, W_mu, b_mu, W_ls, b_ls):
    src = edge_index[0].astype(jnp.int32)
    dst = edge_index[1].astype(jnp.int32)
    ew = edge_weight.astype(jnp.float32)

    pad = EPAD - E
    fill = (jnp.arange(pad, dtype=jnp.int32) * 13) % N
    zfill = jnp.zeros((pad,), jnp.float32)
    srcp = jnp.concatenate([src, fill]).reshape(NW, NCH, K)
    dstp = jnp.concatenate([dst, fill]).reshape(NW, NCH, K)
    ewp = jnp.concatenate([ew, zfill]).reshape(NW, NCH, K)
    pk = jnp.stack([srcp, dstp], axis=2)

    degp = _get_sc_degree()(
        dstp.reshape(NW, DNCH, DK), ewp.reshape(NW, DNCH, DK)
    )
    degp3 = degp[:, :N].reshape(NC, N, 1)

    g0, dis = pl.pallas_call(
        _tc1_body,
        grid=(N // _R,),
        in_specs=[
            pl.BlockSpec((NC, _R, 1), lambda i: (0, i, 0)),
            pl.BlockSpec((_R, C), lambda i: (i, 0)),
            pl.BlockSpec((C, C), lambda i: (0, 0)),
        ],
        out_specs=[
            pl.BlockSpec((_R, C), lambda i: (i, 0)),
            pl.BlockSpec((_R, 1), lambda i: (i, 0)),
        ],
        out_shape=[
            jax.ShapeDtypeStruct((N, C), jnp.float32),
            jax.ShapeDtypeStruct((N, 1), jnp.float32),
        ],
    )(degp3, x, W1)

    p_parts = _get_sc_aggregate()(g0, pk, ewp)

    Wcat = jnp.concatenate([W_mu, W_ls], axis=1)
    bcat = jnp.concatenate([b_mu, b_ls]).reshape(1, C)
    b1r = b1.reshape(1, C)

    g1 = pl.pallas_call(
        _tc2_body,
        grid=(N // _R,),
        in_specs=[
            pl.BlockSpec((NC, _R, C), lambda i: (0, i, 0)),
            pl.BlockSpec((_R, C), lambda i: (i, 0)),
            pl.BlockSpec((_R, 1), lambda i: (i, 0)),
            pl.BlockSpec((1, C), lambda i: (0, 0)),
            pl.BlockSpec((C, C), lambda i: (0, 0)),
        ],
        out_specs=pl.BlockSpec((_R, C), lambda i: (i, 0)),
        out_shape=jax.ShapeDtypeStruct((N, C), jnp.float32),
    )(p_parts, g0, dis, b1r, Wcat)

    q_parts = _get_sc_aggregate()(g1, pk, ewp)

    mu, ls = pl.pallas_call(
        _tc3_body,
        grid=(N // _R,),
        in_specs=[
            pl.BlockSpec((NC, _R, C), lambda i: (0, i, 0)),
            pl.BlockSpec((_R, C), lambda i: (i, 0)),
            pl.BlockSpec((_R, 1), lambda i: (i, 0)),
            pl.BlockSpec((1, C), lambda i: (0, 0)),
        ],
        out_specs=[
            pl.BlockSpec((_R, C // 2), lambda i: (i, 0)),
            pl.BlockSpec((_R, C // 2), lambda i: (i, 0)),
        ],
        out_shape=[
            jax.ShapeDtypeStruct((N, C // 2), jnp.float32),
            jax.ShapeDtypeStruct((N, C // 2), jnp.float32),
        ],
    )(q_parts, g1, dis, bcat)

    return (mu, ls)


# speedup vs baseline: 35.6146x; 1.0479x over previous
import functools

import jax
import jax.numpy as jnp
from jax import lax
from jax.experimental import pallas as pl
from jax.experimental.pallas import tpu as pltpu
from jax.experimental.pallas import tpu_sc as plsc

N = 10000
E = 320000
C = 128
NC, NS = 2, 16
NW = NC * NS
EPW = 10240
EPAD = NW * EPW
K = 80
NCH = EPW // K
NP = 10240
DK = 128
DNCH = EPW // DK
NPR = 10240
RPT = NPR // NS

@functools.lru_cache(maxsize=None)
def _get_mesh():
    return plsc.VectorSubcoreMesh(
        core_axis_name="c", subcore_axis_name="s", num_cores=NC, num_subcores=NS
    )


@functools.lru_cache(maxsize=None)
def _get_sc_degree():
    return functools.partial(
        pl.kernel,
        out_type=jax.ShapeDtypeStruct((NC, NP), jnp.float32),
        mesh=_get_mesh(),
        scratch_types=[
            pltpu.VMEM((DNCH, DK), jnp.int32),
            pltpu.VMEM((DNCH, DK), jnp.float32),
            pltpu.VMEM((NP // NS,), jnp.float32),
            pltpu.MemorySpace.VMEM_SHARED((NP,), jnp.float32),
        ],
    )(_sc_degree_body)


def _sc_degree_body(dst_hbm, ew_hbm, out_hbm, dst_v, ew_v, zero_v, acc_sh):
    c = lax.axis_index("c")
    s = lax.axis_index("s")
    wid = c * NS + s
    zvec = jnp.zeros((16,), jnp.float32)

    def _zero(i, carry):
        zero_v[pl.ds(i * 16, 16)] = zvec
        return carry

    lax.fori_loop(0, (NP // NS) // 16, _zero, 0)
    pltpu.sync_copy(zero_v, acc_sh.at[pl.ds(s * (NP // NS), NP // NS)])
    pltpu.sync_copy(dst_hbm.at[wid], dst_v)
    pltpu.sync_copy(ew_hbm.at[wid], ew_v)
    plsc.subcore_barrier()

    def _chunk(ci, carry):
        pltpu.sync_copy(ew_v.at[ci], acc_sh.at[dst_v.at[ci]], add=True)
        return carry

    lax.fori_loop(0, DNCH, _chunk, 0)
    plsc.subcore_barrier()
    pltpu.sync_copy(
        acc_sh.at[pl.ds(s * (NP // NS), NP // NS)],
        out_hbm.at[c, pl.ds(s * (NP // NS), NP // NS)],
    )


@functools.lru_cache(maxsize=None)
def _get_sc_aggregate():
    return functools.partial(
        pl.kernel,
        out_type=jax.ShapeDtypeStruct((NC, NPR, C), jnp.float32),
        mesh=_get_mesh(),
        scratch_types=[
            pltpu.VMEM((4, 2, K), jnp.int32),
            pltpu.VMEM((4, K), jnp.float32),
            pltpu.VMEM((K, C), jnp.float32),
            pltpu.VMEM((K, C), jnp.float32),
            pltpu.VMEM((K, C), jnp.float32),
            pltpu.VMEM((K, C), jnp.float32),
            pltpu.MemorySpace.VMEM_SHARED((NPR, C), jnp.float32),
            pltpu.SemaphoreType.DMA,
            pltpu.SemaphoreType.DMA,
            pltpu.SemaphoreType.DMA,
            pltpu.SemaphoreType.DMA,
            pltpu.SemaphoreType.DMA,
            pltpu.SemaphoreType.DMA,
            pltpu.SemaphoreType.DMA,
            pltpu.SemaphoreType.DMA,
            pltpu.SemaphoreType.DMA,
            pltpu.SemaphoreType.DMA,
            pltpu.SemaphoreType.DMA,
            pltpu.SemaphoreType.DMA,
        ],
    )(_sc_aggregate_body)


_DNUMS = lax.GatherDimensionNumbers(
    offset_dims=(), collapsed_slice_dims=(0,), start_index_map=(0,)
)


def _sc_aggregate_body(g_hbm, pk_hbm, ew_hbm, out_hbm, pk_v, ew_r,
                       rv0, rv1, rv2, rv3, acc_sh,
                       spk0, spk1, spk2, spk3, sg0, sg1, sg2, sg3,
                       ss0, ss1, ss2, ss3):
    c = lax.axis_index("c")
    s = lax.axis_index("s")
    wid = c * NS + s
    spk = [spk0, spk1, spk2, spk3]
    sg = [sg0, sg1, sg2, sg3]
    ss = [ss0, ss1, ss2, ss3]
    rvs = [rv0, rv1, rv2, rv3]
    zvec = jnp.zeros((16,), jnp.float32)

    def _zero(i, carry):
        for k in range(C // 16):
            rv0[i, pl.ds(k * 16, 16)] = zvec
        return carry

    lax.fori_loop(0, K, _zero, 0)
    for t in range(RPT // K):
        pltpu.sync_copy(rv0, acc_sh.at[pl.ds(s * RPT + t * K, K)])
    plsc.subcore_barrier()

    def _issue_idx(ci, slot):
        pltpu.async_copy(pk_hbm.at[wid, ci], pk_v.at[slot], spk[slot])
        pltpu.async_copy(ew_hbm.at[wid, ci], ew_r.at[slot], spk[slot])

    def _wait_idx(ci, slot):
        pltpu.make_async_copy(
            pk_hbm.at[wid, ci], pk_v.at[slot], spk[slot]
        ).wait()
        pltpu.make_async_copy(
            ew_hbm.at[wid, ci], ew_r.at[slot], spk[slot]
        ).wait()

    _issue_idx(0, 0)
    _wait_idx(0, 0)
    pltpu.async_copy(g_hbm.at[pk_v.at[0, 0]], rv0, sg[0])
    _issue_idx(1, 1)

    def _outer(oi, carry):
        for sl in range(4):
            ci = oi * 4 + sl
            sn1 = (sl + 1) % 4
            sn2 = (sl + 2) % 4
            rv = rvs[sl]

            @pl.when(ci >= 2)
            def _drain_scatter():
                pltpu.make_async_copy(
                    rvs[sn2], acc_sh.at[pk_v.at[sn2, 1]], ss[sn2]
                ).wait()

            @pl.when(ci + 1 < NCH)
            def _issue_next_gather():
                _wait_idx(ci + 1, sn1)
                pltpu.async_copy(g_hbm.at[pk_v.at[sn1, 0]], rvs[sn1], sg[sn1])

            @pl.when(ci + 2 < NCH)
            def _prefetch_idx():
                _issue_idx(ci + 2, sn2)

            pltpu.make_async_copy(g_hbm.at[pk_v.at[sl, 0]], rv, sg[sl]).wait()

            def _group(gi, carry2):
                ew16 = ew_r[sl, pl.ds(gi * 16, 16)]
                for j in range(16):
                    ewj = lax.gather(
                        ew16, jnp.full((16, 1), j, jnp.int32), _DNUMS, (1,),
                        mode=lax.GatherScatterMode.PROMISE_IN_BOUNDS,
                    )
                    row = gi * 16 + j
                    for k in range(C // 16):
                        rv[row, pl.ds(k * 16, 16)] = (
                            rv[row, pl.ds(k * 16, 16)] * ewj
                        )
                return carry2

            lax.fori_loop(0, K // 16, _group, 0)
            pltpu.make_async_copy(
                rv, acc_sh.at[pk_v.at[sl, 1]], ss[sl]
            ).start(add=True)
        return carry

    lax.fori_loop(0, NCH // 4, _outer, 0)
    pltpu.make_async_copy(rv2, acc_sh.at[pk_v.at[2, 1]], ss[2]).wait()
    pltpu.make_async_copy(rv3, acc_sh.at[pk_v.at[3, 1]], ss[3]).wait()
    plsc.subcore_barrier()
    for t in range(RPT // 128):
        pltpu.sync_copy(
            acc_sh.at[pl.ds(s * RPT + t * 128, 128)],
            out_hbm.at[c, pl.ds(s * RPT + t * 128, 128)],
        )


_R = 2000


def _tc1_body(degp_ref, x_ref, w_ref, g0_ref, dis_ref):
    deg = 1.0 + degp_ref[0] + degp_ref[1]
    dis = lax.rsqrt(deg)
    z = jnp.dot(x_ref[...], w_ref[...], preferred_element_type=jnp.float32)
    g0_ref[...] = z * dis
    dis_ref[...] = dis


def _tc2_body(p_ref, g0_ref, dis_ref, b_ref, w_ref, g1_ref):
    dis = dis_ref[...]
    h1 = jnp.maximum(dis * (p_ref[0] + p_ref[1] + g0_ref[...]) + b_ref[...], 0.0)
    m = jnp.dot(h1, w_ref[...], preferred_element_type=jnp.float32)
    g1_ref[...] = m * dis


def _tc3_body(q_ref, g1_ref, dis_ref, b_ref, mu_ref, ls_ref):
    out = dis_ref[...] * (q_ref[0] + q_ref[1] + g1_ref[...]) + b_ref[...]
    mu_ref[...] = out[:, : C // 2]
    ls_ref[...] = out[:, C // 2 :]


def kernel(x, edge_index, edge_weight, W1, b1, W_mu, b_mu, W_ls, b_ls):
    src = edge_index[0].astype(jnp.int32)
    dst = edge_index[1].astype(jnp.int32)
    ew = edge_weight.astype(jnp.float32)

    pad = EPAD - E
    fill = (jnp.arange(pad, dtype=jnp.int32) * 13) % N
    zfill = jnp.zeros((pad,), jnp.float32)
    srcp = jnp.concatenate([src, fill]).reshape(NW, NCH, K)
    dstp = jnp.concatenate([dst, fill]).reshape(NW, NCH, K)
    ewp = jnp.concatenate([ew, zfill]).reshape(NW, NCH, K)
    pk = jnp.stack([srcp, dstp], axis=2)

    degp = _get_sc_degree()(
        dstp.reshape(NW, DNCH, DK), ewp.reshape(NW, DNCH, DK)
    )
    degp3 = degp[:, :N].reshape(NC, N, 1)

    g0, dis = pl.pallas_call(
        _tc1_body,
        grid=(N // _R,),
        in_specs=[
            pl.BlockSpec((NC, _R, 1), lambda i: (0, i, 0)),
            pl.BlockSpec((_R, C), lambda i: (i, 0)),
            pl.BlockSpec((C, C), lambda i: (0, 0)),
        ],
        out_specs=[
            pl.BlockSpec((_R, C), lambda i: (i, 0)),
            pl.BlockSpec((_R, 1), lambda i: (i, 0)),
        ],
        out_shape=[
            jax.ShapeDtypeStruct((N, C), jnp.float32),
            jax.ShapeDtypeStruct((N, 1), jnp.float32),
        ],
    )(degp3, x, W1)

    p_parts = _get_sc_aggregate()(g0, pk, ewp)

    Wcat = jnp.concatenate([W_mu, W_ls], axis=1)
    bcat = jnp.concatenate([b_mu, b_ls]).reshape(1, C)
    b1r = b1.reshape(1, C)

    g1 = pl.pallas_call(
        _tc2_body,
        grid=(N // _R,),
        in_specs=[
            pl.BlockSpec((NC, _R, C), lambda i: (0, i, 0)),
            pl.BlockSpec((_R, C), lambda i: (i, 0)),
            pl.BlockSpec((_R, 1), lambda i: (i, 0)),
            pl.BlockSpec((1, C), lambda i: (0, 0)),
            pl.BlockSpec((C, C), lambda i: (0, 0)),
        ],
        out_specs=pl.BlockSpec((_R, C), lambda i: (i, 0)),
        out_shape=jax.ShapeDtypeStruct((N, C), jnp.float32),
    )(p_parts, g0, dis, b1r, Wcat)

    q_parts = _get_sc_aggregate()(g1, pk, ewp)

    mu, ls = pl.pallas_call(
        _tc3_body,
        grid=(N // _R,),
        in_specs=[
            pl.BlockSpec((NC, _R, C), lambda i: (0, i, 0)),
            pl.BlockSpec((_R, C), lambda i: (i, 0)),
            pl.BlockSpec((_R, 1), lambda i: (i, 0)),
            pl.BlockSpec((1, C), lambda i: (0, 0)),
        ],
        out_specs=[
            pl.BlockSpec((_R, C // 2), lambda i: (i, 0)),
            pl.BlockSpec((_R, C // 2), lambda i: (i, 0)),
        ],
        out_shape=[
            jax.ShapeDtypeStruct((N, C // 2), jnp.float32),
            jax.ShapeDtypeStruct((N, C // 2), jnp.float32),
        ],
    )(q_parts, g1, dis, bcat)

    return (mu, ls)
